# Initial kernel scaffold; baseline (speedup 1.0000x reference)
import jax
import jax.numpy as jnp
from jax.experimental import pallas as pl


def kernel(x, edge_index, W0, b0, g0, be0, W1, b1, g1, be1, W2, b2, g2, be2):
    raise NotImplementedError("write your pallas kernel here")


# speedup vs baseline: 10.7393x; 10.7393x over previous
import functools

import jax
import jax.numpy as jnp
from jax import lax
from jax.experimental import pallas as pl
from jax.experimental.pallas import tpu as pltpu
from jax.experimental.pallas import tpu_sc as plsc

_N = 10000
_D = 128
_E = 320000
_EPS = 1e-5

_NC = 2
_NS = 16
_NW = _NC * _NS
_CHUNK = 128
_KCH = -(-_E // (_NW * _CHUNK))
_EPAD = _NW * _KCH * _CHUNK
_NACC = 10240
_RPT = _NACC // _NS
_NB = 10
_BR = _N // _NB

@functools.cache
def _sc_kernels():
    mesh = plsc.VectorSubcoreMesh(core_axis_name="c", subcore_axis_name="s")

    @functools.partial(
        pl.kernel,
        out_type=jax.ShapeDtypeStruct((_NC, _NACC, _D), jnp.float32),
        mesh=mesh,
        scratch_types=[
            pltpu.VMEM((_CHUNK,), jnp.int32),
            pltpu.VMEM((_CHUNK,), jnp.int32),
            pltpu.VMEM((_CHUNK, _D), jnp.float32),
            pltpu.VMEM_SHARED((_NACC, _D), jnp.float32),
            pltpu.SemaphoreType.DMA,
        ],
    )
    def sc_scatter(hs, src2, dst2, zpad, t, idx_s, idx_d, rows, acc, sem):
        c = lax.axis_index("c")
        s = lax.axis_index("s")
        wid = c * _NS + s
        row0 = s * _RPT
        pltpu.sync_copy(zpad, acc.at[pl.ds(row0, _RPT)])
        plsc.subcore_barrier()

        def chunk(j, carry):
            pltpu.sync_copy(src2.at[wid * _KCH + j], idx_s)
            pltpu.sync_copy(dst2.at[wid * _KCH + j], idx_d)
            pltpu.async_copy(hs.at[idx_s], rows, sem).wait()
            pltpu.sync_copy(rows, acc.at[idx_d], add=True)
            return carry

        lax.fori_loop(0, _KCH, chunk, 0)
        plsc.subcore_barrier()
        pltpu.sync_copy(acc.at[pl.ds(row0, _RPT)], t.at[c, pl.ds(row0, _RPT)])

    return sc_scatter


_MM = dict(preferred_element_type=jnp.float32, precision=lax.Precision.HIGHEST)


def _tc_prep_body(d0, d1, x, w, hs, dinv):
    deg = d0[...] + d1[...] + 1.0
    di = lax.rsqrt(deg)
    dinv[...] = di
    hs[...] = jnp.dot(x[...], w[...], **_MM) * di


def _tc_stats_body(t0, t1, hs, dinv, b, agg, sums):
    i = pl.program_id(0)
    a = (t0[...] + t1[...] + hs[...]) * dinv[...] + b[...]
    agg[...] = a

    @pl.when(i == 0)
    def _():
        sums[...] = jnp.zeros_like(sums)

    sums[0:1, :] += jnp.sum(a, axis=0, keepdims=True)
    sums[1:2, :] += jnp.sum(a * a, axis=0, keepdims=True)


def _bn_relu(agg_ref, sums_ref, g_ref, be_ref):
    mean = sums_ref[0:1, :] * (1.0 / _N)
    var = sums_ref[1:2, :] * (1.0 / _N) - mean * mean
    inv = lax.rsqrt(var + _EPS)
    return jnp.maximum((agg_ref[...] - mean) * inv * g_ref[...] + be_ref[...], 0.0)


def _tc_norm_mm_body(agg, sums, g, be, w, dinv, out):
    y = _bn_relu(agg, sums, g, be)
    out[...] = jnp.dot(y, w[...], **_MM) * dinv[...]


def _tc_norm_final_body(agg, sums, g, be, out):
    out[...] = _bn_relu(agg, sums, g, be)


_blk = lambda r, c: pl.BlockSpec((r, c), lambda i: (i, 0))
_bcast = lambda r, c: pl.BlockSpec((r, c), lambda i: (0, 0))

_tc_prep = pl.pallas_call(
    _tc_prep_body,
    grid=(_NB,),
    in_specs=[_blk(_BR, _D), _blk(_BR, _D), _blk(_BR, _D), _bcast(_D, _D)],
    out_specs=[_blk(_BR, _D), _blk(_BR, _D)],
    out_shape=[
        jax.ShapeDtypeStruct((_N, _D), jnp.float32),
        jax.ShapeDtypeStruct((_N, _D), jnp.float32),
    ],
)

_tc_stats = pl.pallas_call(
    _tc_stats_body,
    grid=(_NB,),
    in_specs=[_blk(_BR, _D), _blk(_BR, _D), _blk(_BR, _D), _blk(_BR, _D),
              _bcast(1, _D)],
    out_specs=[_blk(_BR, _D), _bcast(8, _D)],
    out_shape=[
        jax.ShapeDtypeStruct((_N, _D), jnp.float32),
        jax.ShapeDtypeStruct((8, _D), jnp.float32),
    ],
)

_tc_norm_mm = pl.pallas_call(
    _tc_norm_mm_body,
    grid=(_NB,),
    in_specs=[_blk(_BR, _D), _bcast(8, _D), _bcast(1, _D), _bcast(1, _D),
              _bcast(_D, _D), _blk(_BR, _D)],
    out_specs=_blk(_BR, _D),
    out_shape=jax.ShapeDtypeStruct((_N, _D), jnp.float32),
)

_tc_norm_final = pl.pallas_call(
    _tc_norm_final_body,
    grid=(_NB,),
    in_specs=[_blk(_BR, _D), _bcast(8, _D), _bcast(1, _D), _bcast(1, _D)],
    out_specs=_blk(_BR, _D),
    out_shape=jax.ShapeDtypeStruct((_N, _D), jnp.float32),
)


def kernel(x, edge_index, W0, b0, g0, be0, W1, b1, g1, be1, W2, b2, g2, be2):
    src = edge_index[0]
    dst = edge_index[1]
    pad = _EPAD - _E
    spread = (jnp.arange(pad, dtype=jnp.int32) % 128)
    src2 = jnp.concatenate([src, spread]).reshape(_NW * _KCH, _CHUNK)
    dst2 = jnp.concatenate([dst, _N + spread]).reshape(_NW * _KCH, _CHUNK)
    zpad = jnp.zeros((_RPT, _D), jnp.float32)

    sc_scatter = _sc_kernels()
    ones_nd = jnp.ones((_N, _D), jnp.float32)
    tdeg = sc_scatter(ones_nd, src2, dst2, zpad)
    hs, dinv = _tc_prep(tdeg[0], tdeg[1], x, W0)

    params = ((b0, g0, be0, W1), (b1, g1, be1, W2), (b2, g2, be2, None))
    for b, g, be, Wn in params:
        t = sc_scatter(hs, src2, dst2, zpad)
        agg, sums = _tc_stats(t[0], t[1], hs, dinv, b.reshape(1, _D))
        if Wn is not None:
            hs = _tc_norm_mm(agg, sums, g.reshape(1, _D), be.reshape(1, _D),
                             Wn, dinv)
        else:
            out = _tc_norm_final(agg, sums, g.reshape(1, _D), be.reshape(1, _D))
    return out


# speedup vs baseline: 20.3065x; 1.8909x over previous
import functools

import jax
import jax.numpy as jnp
from jax import lax
from jax.experimental import pallas as pl
from jax.experimental.pallas import tpu as pltpu
from jax.experimental.pallas import tpu_sc as plsc

_N = 10000
_D = 128
_E = 320000
_EPS = 1e-5

_NC = 2
_NS = 16
_NW = _NC * _NS
_CHUNK = 128
_KCH = 80
_PH = 40
_EPAD = _NW * _KCH * _CHUNK
_NACC = 10240
_RPT = _NACC // _NS
_NB = 10
_BR = _N // _NB

@functools.cache
def _sc_kernels():
    mesh = plsc.VectorSubcoreMesh(core_axis_name="c", subcore_axis_name="s")

    @functools.partial(
        pl.kernel,
        out_type=jax.ShapeDtypeStruct((_NC, _NACC, _D), jnp.float32),
        mesh=mesh,
        scratch_types=[
            pltpu.VMEM((_PH, _CHUNK), jnp.int32),
            pltpu.VMEM((_PH, _CHUNK), jnp.int32),
            pltpu.VMEM((2, _CHUNK, _D), jnp.float32),
            pltpu.VMEM_SHARED((_NACC, _D), jnp.float32),
            pltpu.SemaphoreType.DMA,
            pltpu.SemaphoreType.DMA,
        ],
    )
    def sc_scatter(hs, src2, dst2, zpad, t, isa, ida, rows2, acc, g0, g1):
        c = lax.axis_index("c")
        s = lax.axis_index("s")
        wid = c * _NS + s
        row0 = s * _RPT
        pltpu.sync_copy(zpad, acc.at[pl.ds(row0, _RPT)])
        plsc.subcore_barrier()

        for p in range(_KCH // _PH):
            base = wid * _KCH + p * _PH
            pltpu.sync_copy(src2.at[pl.ds(base, _PH)], isa)
            pltpu.sync_copy(dst2.at[pl.ds(base, _PH)], ida)
            pltpu.async_copy(hs.at[isa.at[0]], rows2.at[0], g0)

            def body2(i, carry):
                j = 2 * i
                pltpu.async_copy(hs.at[isa.at[j + 1]], rows2.at[1], g1)
                pltpu.make_async_copy(hs.at[isa.at[0]], rows2.at[0], g0).wait()
                pltpu.sync_copy(rows2.at[0], acc.at[ida.at[j]], add=True)
                pltpu.async_copy(hs.at[isa.at[j + 2]], rows2.at[0], g0)
                pltpu.make_async_copy(hs.at[isa.at[0]], rows2.at[1], g1).wait()
                pltpu.sync_copy(rows2.at[1], acc.at[ida.at[j + 1]], add=True)
                return carry

            lax.fori_loop(0, _PH // 2 - 1, body2, 0)
            pltpu.async_copy(hs.at[isa.at[_PH - 1]], rows2.at[1], g1)
            pltpu.make_async_copy(hs.at[isa.at[0]], rows2.at[0], g0).wait()
            pltpu.sync_copy(rows2.at[0], acc.at[ida.at[_PH - 2]], add=True)
            pltpu.make_async_copy(hs.at[isa.at[0]], rows2.at[1], g1).wait()
            pltpu.sync_copy(rows2.at[1], acc.at[ida.at[_PH - 1]], add=True)

        plsc.subcore_barrier()
        pltpu.sync_copy(acc.at[pl.ds(row0, _RPT)], t.at[c, pl.ds(row0, _RPT)])

    return sc_scatter


_MM = dict(preferred_element_type=jnp.float32, precision=lax.Precision.HIGHEST)


def _tc_prep_body(d0, d1, x, w, hs, dinv):
    deg = d0[...] + d1[...] + 1.0
    di = lax.rsqrt(deg)
    dinv[...] = di
    hs[...] = jnp.dot(x[...], w[...], **_MM) * di


def _tc_stats_body(t0, t1, hs, dinv, b, agg, sums):
    i = pl.program_id(0)
    a = (t0[...] + t1[...] + hs[...]) * dinv[...] + b[...]
    agg[...] = a

    @pl.when(i == 0)
    def _():
        sums[...] = jnp.zeros_like(sums)

    sums[0:1, :] += jnp.sum(a, axis=0, keepdims=True)
    sums[1:2, :] += jnp.sum(a * a, axis=0, keepdims=True)


def _bn_relu(agg_ref, sums_ref, g_ref, be_ref):
    mean = sums_ref[0:1, :] * (1.0 / _N)
    var = sums_ref[1:2, :] * (1.0 / _N) - mean * mean
    inv = lax.rsqrt(var + _EPS)
    return jnp.maximum((agg_ref[...] - mean) * inv * g_ref[...] + be_ref[...], 0.0)


def _tc_norm_mm_body(agg, sums, g, be, w, dinv, out):
    y = _bn_relu(agg, sums, g, be)
    out[...] = jnp.dot(y, w[...], **_MM) * dinv[...]


def _tc_norm_final_body(agg, sums, g, be, out):
    out[...] = _bn_relu(agg, sums, g, be)


_blk = lambda r, c: pl.BlockSpec((r, c), lambda i: (i, 0))
_bcast = lambda r, c: pl.BlockSpec((r, c), lambda i: (0, 0))

_tc_prep = pl.pallas_call(
    _tc_prep_body,
    grid=(_NB,),
    in_specs=[_blk(_BR, _D), _blk(_BR, _D), _blk(_BR, _D), _bcast(_D, _D)],
    out_specs=[_blk(_BR, _D), _blk(_BR, _D)],
    out_shape=[
        jax.ShapeDtypeStruct((_N, _D), jnp.float32),
        jax.ShapeDtypeStruct((_N, _D), jnp.float32),
    ],
)

_tc_stats = pl.pallas_call(
    _tc_stats_body,
    grid=(_NB,),
    in_specs=[_blk(_BR, _D), _blk(_BR, _D), _blk(_BR, _D), _blk(_BR, _D),
              _bcast(1, _D)],
    out_specs=[_blk(_BR, _D), _bcast(8, _D)],
    out_shape=[
        jax.ShapeDtypeStruct((_N, _D), jnp.float32),
        jax.ShapeDtypeStruct((8, _D), jnp.float32),
    ],
)

_tc_norm_mm = pl.pallas_call(
    _tc_norm_mm_body,
    grid=(_NB,),
    in_specs=[_blk(_BR, _D), _bcast(8, _D), _bcast(1, _D), _bcast(1, _D),
              _bcast(_D, _D), _blk(_BR, _D)],
    out_specs=_blk(_BR, _D),
    out_shape=jax.ShapeDtypeStruct((_N, _D), jnp.float32),
)

_tc_norm_final = pl.pallas_call(
    _tc_norm_final_body,
    grid=(_NB,),
    in_specs=[_blk(_BR, _D), _bcast(8, _D), _bcast(1, _D), _bcast(1, _D)],
    out_specs=_blk(_BR, _D),
    out_shape=jax.ShapeDtypeStruct((_N, _D), jnp.float32),
)


def kernel(x, edge_index, W0, b0, g0, be0, W1, b1, g1, be1, W2, b2, g2, be2):
    src = edge_index[0]
    dst = edge_index[1]
    pad = _EPAD - _E
    spread = (jnp.arange(pad, dtype=jnp.int32) % 128)
    src2 = jnp.concatenate([src, spread]).reshape(_NW * _KCH, _CHUNK)
    dst2 = jnp.concatenate([dst, _N + spread]).reshape(_NW * _KCH, _CHUNK)
    zpad = jnp.zeros((_RPT, _D), jnp.float32)

    sc_scatter = _sc_kernels()
    ones_nd = jnp.ones((_N, _D), jnp.float32)
    tdeg = sc_scatter(ones_nd, src2, dst2, zpad)
    hs, dinv = _tc_prep(tdeg[0], tdeg[1], x, W0)

    params = ((b0, g0, be0, W1), (b1, g1, be1, W2), (b2, g2, be2, None))
    for b, g, be, Wn in params:
        t = sc_scatter(hs, src2, dst2, zpad)
        agg, sums = _tc_stats(t[0], t[1], hs, dinv, b.reshape(1, _D))
        if Wn is not None:
            hs = _tc_norm_mm(agg, sums, g.reshape(1, _D), be.reshape(1, _D),
                             Wn, dinv)
        else:
            out = _tc_norm_final(agg, sums, g.reshape(1, _D), be.reshape(1, _D))
    return out


# speedup vs baseline: 21.6007x; 1.0637x over previous
import functools

import jax
import jax.numpy as jnp
from jax import lax
from jax.experimental import pallas as pl
from jax.experimental.pallas import tpu as pltpu
from jax.experimental.pallas import tpu_sc as plsc

_N = 10000
_D = 128
_E = 320000
_EPS = 1e-5

_NC = 2
_NS = 16
_NW = _NC * _NS
_CHUNK = 128
_KCH = 80
_PH = 40
_EPAD = _NW * _KCH * _CHUNK
_NACC = 10240
_RPT = _NACC // _NS
_NB = 10
_BR = _N // _NB

@functools.cache
def _sc_kernels():
    mesh = plsc.VectorSubcoreMesh(core_axis_name="c", subcore_axis_name="s")

    @functools.partial(
        pl.kernel,
        out_type=jax.ShapeDtypeStruct((_NC, _NACC, _D), jnp.float32),
        mesh=mesh,
        scratch_types=[
            pltpu.VMEM((_PH, _CHUNK), jnp.int32),
            pltpu.VMEM((_PH, _CHUNK), jnp.int32),
            pltpu.VMEM((2, _CHUNK, _D), jnp.float32),
            pltpu.VMEM_SHARED((_NACC, _D), jnp.float32),
            pltpu.SemaphoreType.DMA,
            pltpu.SemaphoreType.DMA,
        ],
    )
    def sc_scatter(hs, src2, dst2, zpad, t, isa, ida, rows2, acc, g0, g1):
        c = lax.axis_index("c")
        s = lax.axis_index("s")
        wid = c * _NS + s
        row0 = s * _RPT
        pltpu.sync_copy(zpad, acc.at[pl.ds(row0, _RPT)])
        plsc.subcore_barrier()

        for p in range(_KCH // _PH):
            base = wid * _KCH + p * _PH
            pltpu.sync_copy(src2.at[pl.ds(base, _PH)], isa)
            pltpu.sync_copy(dst2.at[pl.ds(base, _PH)], ida)
            pltpu.async_copy(hs.at[isa.at[0]], rows2.at[0], g0)

            def body2(i, carry):
                j = 2 * i
                pltpu.async_copy(hs.at[isa.at[j + 1]], rows2.at[1], g1)
                pltpu.make_async_copy(hs.at[isa.at[0]], rows2.at[0], g0).wait()
                pltpu.sync_copy(rows2.at[0], acc.at[ida.at[j]], add=True)
                pltpu.async_copy(hs.at[isa.at[j + 2]], rows2.at[0], g0)
                pltpu.make_async_copy(hs.at[isa.at[0]], rows2.at[1], g1).wait()
                pltpu.sync_copy(rows2.at[1], acc.at[ida.at[j + 1]], add=True)
                return carry

            lax.fori_loop(0, _PH // 2 - 1, body2, 0)
            pltpu.async_copy(hs.at[isa.at[_PH - 1]], rows2.at[1], g1)
            pltpu.make_async_copy(hs.at[isa.at[0]], rows2.at[0], g0).wait()
            pltpu.sync_copy(rows2.at[0], acc.at[ida.at[_PH - 2]], add=True)
            pltpu.make_async_copy(hs.at[isa.at[0]], rows2.at[1], g1).wait()
            pltpu.sync_copy(rows2.at[1], acc.at[ida.at[_PH - 1]], add=True)

        plsc.subcore_barrier()
        pltpu.sync_copy(acc.at[pl.ds(row0, _RPT)], t.at[c, pl.ds(row0, _RPT)])

    @functools.partial(
        pl.kernel,
        out_type=jax.ShapeDtypeStruct((_NC, _NACC, _D), jnp.float32),
        mesh=mesh,
        scratch_types=[
            pltpu.VMEM((_PH, _CHUNK), jnp.int32),
            pltpu.VMEM((_CHUNK, _D), jnp.float32),
            pltpu.VMEM_SHARED((_NACC, _D), jnp.float32),
            pltpu.SemaphoreType.DMA,
        ],
    )
    def sc_count(ones_nd, dst2, zpad, tdeg, ida, rows, acc, sem):
        c = lax.axis_index("c")
        s = lax.axis_index("s")
        wid = c * _NS + s
        row0 = s * _RPT
        pltpu.sync_copy(ones_nd.at[pl.ds(0, _CHUNK)], rows)
        pltpu.sync_copy(zpad, acc.at[pl.ds(row0, _RPT)])
        plsc.subcore_barrier()

        for p in range(_KCH // _PH):
            pltpu.sync_copy(dst2.at[pl.ds(wid * _KCH + p * _PH, _PH)], ida)

            def fire(j, carry):
                pltpu.async_copy(rows, acc.at[ida.at[j]], sem, add=True)
                return carry

            lax.fori_loop(0, _PH, fire, 0)

            def drain(j, carry):
                pltpu.make_async_copy(rows, acc.at[ida.at[0]], sem).wait()
                return carry

            lax.fori_loop(0, _PH, drain, 0)

        plsc.subcore_barrier()
        pltpu.sync_copy(acc.at[pl.ds(row0, _RPT)], tdeg.at[c, pl.ds(row0, _RPT)])

    return sc_scatter, sc_count


_MM = dict(preferred_element_type=jnp.float32, precision=lax.Precision.HIGHEST)


def _tc_prep_body(d0, d1, x, w, hs, dinv):
    deg = d0[...] + d1[...] + 1.0
    di = lax.rsqrt(deg)
    dinv[...] = di
    hs[...] = jnp.dot(x[...], w[...], **_MM) * di


def _tc_stats_body(t0, t1, hs, dinv, b, agg, sums):
    i = pl.program_id(0)
    a = (t0[...] + t1[...] + hs[...]) * dinv[...] + b[...]
    agg[...] = a

    @pl.when(i == 0)
    def _():
        sums[...] = jnp.zeros_like(sums)

    sums[0:1, :] += jnp.sum(a, axis=0, keepdims=True)
    sums[1:2, :] += jnp.sum(a * a, axis=0, keepdims=True)


def _bn_relu(agg_ref, sums_ref, g_ref, be_ref):
    mean = sums_ref[0:1, :] * (1.0 / _N)
    var = sums_ref[1:2, :] * (1.0 / _N) - mean * mean
    inv = lax.rsqrt(var + _EPS)
    return jnp.maximum((agg_ref[...] - mean) * inv * g_ref[...] + be_ref[...], 0.0)


def _tc_norm_mm_body(agg, sums, g, be, w, dinv, out):
    y = _bn_relu(agg, sums, g, be)
    out[...] = jnp.dot(y, w[...], **_MM) * dinv[...]


def _tc_norm_final_body(agg, sums, g, be, out):
    out[...] = _bn_relu(agg, sums, g, be)


_blk = lambda r, c: pl.BlockSpec((r, c), lambda i: (i, 0))
_bcast = lambda r, c: pl.BlockSpec((r, c), lambda i: (0, 0))

_tc_prep = pl.pallas_call(
    _tc_prep_body,
    grid=(_NB,),
    in_specs=[_blk(_BR, _D), _blk(_BR, _D), _blk(_BR, _D), _bcast(_D, _D)],
    out_specs=[_blk(_BR, _D), _blk(_BR, _D)],
    out_shape=[
        jax.ShapeDtypeStruct((_N, _D), jnp.float32),
        jax.ShapeDtypeStruct((_N, _D), jnp.float32),
    ],
)

_tc_stats = pl.pallas_call(
    _tc_stats_body,
    grid=(_NB,),
    in_specs=[_blk(_BR, _D), _blk(_BR, _D), _blk(_BR, _D), _blk(_BR, _D),
              _bcast(1, _D)],
    out_specs=[_blk(_BR, _D), _bcast(8, _D)],
    out_shape=[
        jax.ShapeDtypeStruct((_N, _D), jnp.float32),
        jax.ShapeDtypeStruct((8, _D), jnp.float32),
    ],
)

_tc_norm_mm = pl.pallas_call(
    _tc_norm_mm_body,
    grid=(_NB,),
    in_specs=[_blk(_BR, _D), _bcast(8, _D), _bcast(1, _D), _bcast(1, _D),
              _bcast(_D, _D), _blk(_BR, _D)],
    out_specs=_blk(_BR, _D),
    out_shape=jax.ShapeDtypeStruct((_N, _D), jnp.float32),
)

_tc_norm_final = pl.pallas_call(
    _tc_norm_final_body,
    grid=(_NB,),
    in_specs=[_blk(_BR, _D), _bcast(8, _D), _bcast(1, _D), _bcast(1, _D)],
    out_specs=_blk(_BR, _D),
    out_shape=jax.ShapeDtypeStruct((_N, _D), jnp.float32),
)


def kernel(x, edge_index, W0, b0, g0, be0, W1, b1, g1, be1, W2, b2, g2, be2):
    src = edge_index[0]
    dst = edge_index[1]
    pad = _EPAD - _E
    spread = (jnp.arange(pad, dtype=jnp.int32) % 128)
    src2 = jnp.concatenate([src, spread]).reshape(_NW * _KCH, _CHUNK)
    dst2 = jnp.concatenate([dst, _N + spread]).reshape(_NW * _KCH, _CHUNK)
    zpad = jnp.zeros((_RPT, _D), jnp.float32)

    sc_scatter, sc_count = _sc_kernels()
    ones_nd = jnp.ones((_N, _D), jnp.float32)
    tdeg = sc_count(ones_nd, dst2, zpad)
    hs, dinv = _tc_prep(tdeg[0], tdeg[1], x, W0)

    params = ((b0, g0, be0, W1), (b1, g1, be1, W2), (b2, g2, be2, None))
    for b, g, be, Wn in params:
        t = sc_scatter(hs, src2, dst2, zpad)
        agg, sums = _tc_stats(t[0], t[1], hs, dinv, b.reshape(1, _D))
        if Wn is not None:
            hs = _tc_norm_mm(agg, sums, g.reshape(1, _D), be.reshape(1, _D),
                             Wn, dinv)
        else:
            out = _tc_norm_final(agg, sums, g.reshape(1, _D), be.reshape(1, _D))
    return out
